# baseline (speedup 1.0000x reference)
import jax
import jax.numpy as jnp
from jax import lax
from jax.experimental import pallas as pl
from jax.experimental.pallas import tpu as pltpu
from jax.experimental.pallas import tpu_sc as plsc

B = 16384
D = 64
W = 2 * D
K = 20
NC = 2
NS = 16
NW = NC * NS
L = 16
Q = D // L
PER_W = B // NW
C = 16
NCHUNK = PER_W // C
IO_C = 2 * C
NEG_C = C * K
TP = 17
NT = K + 1


def _log_sigmoid(x):
    u = jnp.exp(-jnp.abs(x))
    z = u / (u + 2.0)
    z2 = z * z
    poly = 1.0 + z2 * (1.0 / 3.0 + z2 * (1.0 / 5.0 + z2 * (1.0 / 7.0 + z2 * (1.0 / 9.0))))
    return jnp.minimum(x, 0.0) - 2.0 * z * poly


def _sc_body(io_idx_hbm, neg_idx_hbm, tab,
             out_hbm, pio_v, pneg_v,
             iob0, negb0, iob1, negb1,
             tbuf, partial_v, sem0, sem1):
    wid = lax.axis_index("s") * NC + lax.axis_index("c")
    base = wid * PER_W

    pltpu.sync_copy(io_idx_hbm.at[pl.ds(base * 2, PER_W * 2)], pio_v)
    pltpu.sync_copy(neg_idx_hbm.at[pl.ds(base * K, PER_W * K)], pneg_v)

    lane = lax.iota(jnp.int32, L)
    zeros = jnp.zeros((L,), jnp.float32)
    laneq_in = [lane + q * L for q in range(Q)]
    laneq_out = [lane + D + q * L for q in range(Q)]
    for i in range(2 * L * TP // L):
        tbuf[pl.ds(i * L, L)] = zeros

    sign1 = jnp.where(lane == 0, 1.0, -1.0)
    mask2 = jnp.where(lane < NT - L, 1.0, 0.0)

    def issue(c, iob, negb, sem):
        pltpu.async_copy(tab.at[pio_v.at[pl.ds(c * IO_C, IO_C)]], iob, sem)
        nb = c * NEG_C
        pltpu.async_copy(tab.at[pneg_v.at[pl.ds(nb, 128)]],
                         negb.at[pl.ds(0, 128)], sem)
        pltpu.async_copy(tab.at[pneg_v.at[pl.ds(nb + 128, 128)]],
                         negb.at[pl.ds(128, 128)], sem)
        pltpu.async_copy(tab.at[pneg_v.at[pl.ds(nb + 256, 64)]],
                         negb.at[pl.ds(256, 64)], sem)

    def drain(iob, negb, sem):
        pltpu.make_async_copy(tab.at[pio_v.at[pl.ds(0, IO_C)]], iob, sem).wait()
        pltpu.make_async_copy(tab.at[pneg_v.at[pl.ds(0, 128)]],
                              negb.at[pl.ds(0, 128)], sem).wait()
        pltpu.make_async_copy(tab.at[pneg_v.at[pl.ds(128, 128)]],
                              negb.at[pl.ds(128, 128)], sem).wait()
        pltpu.make_async_copy(tab.at[pneg_v.at[pl.ds(256, 64)]],
                              negb.at[pl.ds(256, 64)], sem).wait()

    def compute(iob, negb, total):
        def elem(e, tot):
            erow = jnp.full((L,), e, jnp.int32)
            in_q = [plsc.load_gather(iob, [erow, laneq_in[q]])
                    for q in range(Q)]
            for t in range(NT):
                if t == 0:
                    row = iob
                    rowv = erow + C
                else:
                    row = negb
                    rowv = jnp.full((L,), e * K + (t - 1), jnp.int32)
                p = in_q[0] * plsc.load_gather(row, [rowv, laneq_out[0]])
                for q in range(1, Q):
                    p = p + in_q[q] * plsc.load_gather(row, [rowv, laneq_out[q]])
                plsc.store_scatter(tbuf, [lane + t * TP], p)
            r1 = plsc.load_gather(tbuf, [lane * TP])
            r2 = plsc.load_gather(tbuf, [lane * TP + L * TP])
            for j in range(1, L):
                r1 = r1 + plsc.load_gather(tbuf, [lane * TP + j])
                r2 = r2 + plsc.load_gather(tbuf, [lane * TP + L * TP + j])
            return tot + _log_sigmoid(r1 * sign1) + _log_sigmoid(-r2) * mask2

        return lax.fori_loop(0, C, elem, total)

    issue(0, iob0, negb0, sem0)

    def pair(i, total):
        c0 = 2 * i
        issue(c0 + 1, iob1, negb1, sem1)
        drain(iob0, negb0, sem0)
        total = compute(iob0, negb0, total)
        cn = lax.select(c0 + 2 < NCHUNK, c0 + 2, 0)
        issue(cn, iob0, negb0, sem0)
        drain(iob1, negb1, sem1)
        return compute(iob1, negb1, total)

    total = lax.fori_loop(0, NCHUNK // 2, pair, zeros)
    drain(iob0, negb0, sem0)

    partial_v[...] = total * (1.0 / B)
    pltpu.sync_copy(partial_v, out_hbm.at[pl.ds(wid * L, L)])


@jax.jit
def _sc_call(io_idx, neg_flat, tab):
    mesh = plsc.VectorSubcoreMesh(core_axis_name="c", subcore_axis_name="s")
    f = pl.kernel(
        _sc_body,
        out_type=jax.ShapeDtypeStruct((NW * L,), jnp.float32),
        mesh=mesh,
        scratch_types=[
            pltpu.VMEM((2 * PER_W,), jnp.int32),
            pltpu.VMEM((PER_W * K,), jnp.int32),
            pltpu.VMEM((IO_C, W), jnp.float32),
            pltpu.VMEM((NEG_C, W), jnp.float32),
            pltpu.VMEM((IO_C, W), jnp.float32),
            pltpu.VMEM((NEG_C, W), jnp.float32),
            pltpu.VMEM((2 * L * TP,), jnp.float32),
            pltpu.VMEM((L,), jnp.float32),
            pltpu.SemaphoreType.DMA,
            pltpu.SemaphoreType.DMA,
        ],
        compiler_params=pltpu.CompilerParams(
            needs_layout_passes=False, use_tc_tiling_on_sc=True),
    )
    return f(io_idx, neg_flat, tab)


def kernel(input_idx, output_idx, neg_idx, input_vectors, output_vectors):
    ii = input_idx.astype(jnp.int32)
    oi = output_idx.astype(jnp.int32)
    ni = neg_idx.astype(jnp.int32)
    iv, ov = lax.optimization_barrier((input_vectors, output_vectors))
    tab = jnp.concatenate([iv, ov], axis=1)
    io_idx = jnp.concatenate(
        [ii.reshape(-1, C), oi.reshape(-1, C)], axis=1).reshape(-1)
    partials = _sc_call(io_idx, ni.reshape(-1), tab)
    return jnp.sum(partials)


# speedup vs baseline: 2.2373x; 2.2373x over previous
import jax
import jax.numpy as jnp
from jax import lax
from jax.experimental import pallas as pl
from jax.experimental.pallas import tpu as pltpu
from jax.experimental.pallas import tpu_sc as plsc

B = 16384
D = 64
W = 2 * D
K = 20
NC = 2
NS = 16
NW = NC * NS
L = 16
Q = D // L
PER_W = B // NW
C = 16
NCHUNK = PER_W // C
IO_C = 2 * C
NEG_C = C * K
TP = 17
NT = K + 1


def _log_sigmoid(x):
    u = jnp.exp(-jnp.abs(x))
    z = u / (u + 2.0)
    z2 = z * z
    poly = 1.0 + z2 * (1.0 / 3.0 + z2 * (1.0 / 5.0 + z2 * (1.0 / 7.0 + z2 * (1.0 / 9.0))))
    return jnp.minimum(x, 0.0) - 2.0 * z * poly


def _sc_body(io_idx_hbm, neg_idx_hbm, tab,
             out_hbm, pio_v, pneg_v,
             iob0, negb0, iob1, negb1,
             tbuf, partial_v, sem0, sem1):
    wid = lax.axis_index("s") * NC + lax.axis_index("c")
    base = wid * PER_W

    pltpu.sync_copy(io_idx_hbm.at[pl.ds(base * 2, PER_W * 2)], pio_v)
    pltpu.sync_copy(neg_idx_hbm.at[pl.ds(base * K, PER_W * K)], pneg_v)

    lane = lax.iota(jnp.int32, L)
    zeros = jnp.zeros((L,), jnp.float32)
    laneq_in = [lane + q * L for q in range(Q)]
    laneq_out = [lane + D + q * L for q in range(Q)]
    for i in range(2 * L * TP // L):
        tbuf[pl.ds(i * L, L)] = zeros

    sign1 = jnp.where(lane == 0, 1.0, -1.0)
    mask2 = jnp.where(lane < NT - L, 1.0, 0.0)

    def issue(c, iob, negb, sem):
        pltpu.async_copy(tab.at[pio_v.at[pl.ds(c * IO_C, IO_C)]], iob, sem)
        nb = c * NEG_C
        pltpu.async_copy(tab.at[pneg_v.at[pl.ds(nb, 128)]],
                         negb.at[pl.ds(0, 128)], sem)
        pltpu.async_copy(tab.at[pneg_v.at[pl.ds(nb + 128, 128)]],
                         negb.at[pl.ds(128, 128)], sem)
        pltpu.async_copy(tab.at[pneg_v.at[pl.ds(nb + 256, 64)]],
                         negb.at[pl.ds(256, 64)], sem)

    def drain(iob, negb, sem):
        pltpu.make_async_copy(tab.at[pio_v.at[pl.ds(0, IO_C)]], iob, sem).wait()
        pltpu.make_async_copy(tab.at[pneg_v.at[pl.ds(0, 128)]],
                              negb.at[pl.ds(0, 128)], sem).wait()
        pltpu.make_async_copy(tab.at[pneg_v.at[pl.ds(128, 128)]],
                              negb.at[pl.ds(128, 128)], sem).wait()
        pltpu.make_async_copy(tab.at[pneg_v.at[pl.ds(256, 64)]],
                              negb.at[pl.ds(256, 64)], sem).wait()

    def compute(iob, negb, total):
        def elem(e, tot):
            erow = jnp.full((L,), e, jnp.int32)
            in_q = [plsc.load_gather(iob, [erow, laneq_in[q]])
                    for q in range(Q)]
            for t in range(NT):
                if t == 0:
                    row = iob
                    rowv = erow + C
                else:
                    row = negb
                    rowv = jnp.full((L,), e * K + (t - 1), jnp.int32)
                p = in_q[0] * plsc.load_gather(row, [rowv, laneq_out[0]])
                for q in range(1, Q):
                    p = p + in_q[q] * plsc.load_gather(row, [rowv, laneq_out[q]])
                plsc.store_scatter(tbuf, [lane + t * TP], p)
            r1 = plsc.load_gather(tbuf, [lane * TP])
            r2 = plsc.load_gather(tbuf, [lane * TP + L * TP])
            for j in range(1, L):
                r1 = r1 + plsc.load_gather(tbuf, [lane * TP + j])
                r2 = r2 + plsc.load_gather(tbuf, [lane * TP + L * TP + j])
            return tot + _log_sigmoid(r1 * sign1) + _log_sigmoid(-r2) * mask2

        return lax.fori_loop(0, C, elem, total)

    issue(0, iob0, negb0, sem0)

    def pair(i, total):
        c0 = 2 * i
        issue(c0 + 1, iob1, negb1, sem1)
        drain(iob0, negb0, sem0)
        total = compute(iob0, negb0, total)
        cn = lax.select(c0 + 2 < NCHUNK, c0 + 2, 0)
        issue(cn, iob0, negb0, sem0)
        drain(iob1, negb1, sem1)
        return compute(iob1, negb1, total)

    total = lax.fori_loop(0, NCHUNK // 2, pair, zeros)
    drain(iob0, negb0, sem0)

    partial_v[...] = total * (1.0 / B)
    pltpu.sync_copy(partial_v, out_hbm.at[pl.ds(wid * L, L)])


TBLK = 20480
NROWS = 1000000


def _tc_combine_body(a_ref, b_ref, o_ref):
    eye = jnp.float32(
        lax.broadcasted_iota(jnp.int32, (D, D), 0)
        == lax.broadcasted_iota(jnp.int32, (D, D), 1))
    dn = (((0,), (0,)), ((), ()))
    o_ref[:, 0:D] = lax.dot_general(
        a_ref[...], eye, dn, preferred_element_type=jnp.float32)
    o_ref[:, D:W] = lax.dot_general(
        b_ref[...], eye, dn, preferred_element_type=jnp.float32)


def _tc_combine(inT, outT):
    grid = (pl.cdiv(NROWS, TBLK),)
    in_spec = pl.BlockSpec((D, TBLK), lambda i: (0, i))
    out_spec = pl.BlockSpec((TBLK, W), lambda i: (i, 0))
    return pl.pallas_call(
        _tc_combine_body,
        grid=grid,
        in_specs=[in_spec, in_spec],
        out_specs=out_spec,
        out_shape=jax.ShapeDtypeStruct((NROWS, W), jnp.float32),
    )(inT, outT)


@jax.jit
def _sc_call(io_idx, neg_flat, tab):
    mesh = plsc.VectorSubcoreMesh(core_axis_name="c", subcore_axis_name="s")
    f = pl.kernel(
        _sc_body,
        out_type=jax.ShapeDtypeStruct((NW * L,), jnp.float32),
        mesh=mesh,
        scratch_types=[
            pltpu.VMEM((2 * PER_W,), jnp.int32),
            pltpu.VMEM((PER_W * K,), jnp.int32),
            pltpu.VMEM((IO_C, W), jnp.float32),
            pltpu.VMEM((NEG_C, W), jnp.float32),
            pltpu.VMEM((IO_C, W), jnp.float32),
            pltpu.VMEM((NEG_C, W), jnp.float32),
            pltpu.VMEM((2 * L * TP,), jnp.float32),
            pltpu.VMEM((L,), jnp.float32),
            pltpu.SemaphoreType.DMA,
            pltpu.SemaphoreType.DMA,
        ],
        compiler_params=pltpu.CompilerParams(
            needs_layout_passes=False, use_tc_tiling_on_sc=True),
    )
    return f(io_idx, neg_flat, tab)


def kernel(input_idx, output_idx, neg_idx, input_vectors, output_vectors):
    ii = input_idx.astype(jnp.int32)
    oi = output_idx.astype(jnp.int32)
    ni = neg_idx.astype(jnp.int32)
    tab = _tc_combine(input_vectors.T, output_vectors.T)
    io_idx = jnp.concatenate(
        [ii.reshape(-1, C), oi.reshape(-1, C)], axis=1).reshape(-1)
    partials = _sc_call(io_idx, ni.reshape(-1), tab)
    return jnp.sum(partials)
